# baseline (speedup 1.0000x reference)
import jax
import jax.numpy as jnp
from jax import lax
from jax.experimental import pallas as pl
from jax.experimental.pallas import tpu as pltpu
from jax.experimental.pallas import tpu_sc as plsc

N = 10000
E = 320000
D = 128
L = 16
NC = 2
NS = 16
NW = NC * NS
EPW = E // NW
B = 100
CHUNKS = EPW // B
ZB = 80
NBUF = 2
N_PAD = 10240
ROWS_PT = N_PAD // NS


def _sc_edge_kernel(x_hbm, src_hbm, dst_hbm, ew_hbm, out_hbm,
                    src_i, dst_i, ewb, xb, agg,
                    gs, es, ss, isrc, idst):
    cid = lax.axis_index("c")
    sid = lax.axis_index("s")
    wid = cid * NS + sid
    ebase = wid * EPW

    zero = jnp.zeros((L,), jnp.float32)

    def zrow(r, _):
        for j in range(D // L):
            ewb[0][r, pl.ds(j * L, L)] = zero
        return 0

    lax.fori_loop(0, ZB, zrow, 0)
    for k in range(ROWS_PT // ZB):
        pltpu.sync_copy(ewb[0].at[pl.ds(0, ZB)],
                        agg.at[pl.ds(sid * ROWS_PT + k * ZB, ZB)])
    plsc.subcore_barrier()

    def issue_src(c, p):
        pltpu.async_copy(src_hbm.at[wid * CHUNKS + c], src_i[p], isrc[p])

    def issue_dst(c, p):
        pltpu.async_copy(dst_hbm.at[wid * CHUNKS + c], dst_i[p], idst[p])

    def issue_gather(p):
        pltpu.async_copy(x_hbm.at[src_i[p]], xb[p], gs[p])

    def issue_ew(c, p):
        pltpu.async_copy(ew_hbm.at[pl.ds(ebase + c * B, B)], ewb[p], es[p])

    def wait_src(p):
        pltpu.make_async_copy(src_hbm.at[0], src_i[p], isrc[p]).wait()

    def wait_dst(p):
        pltpu.make_async_copy(dst_hbm.at[0], dst_i[p], idst[p]).wait()

    def wait_gather_ew(p):
        pltpu.make_async_copy(x_hbm.at[src_i[p]], xb[p], gs[p]).wait()
        pltpu.make_async_copy(ew_hbm.at[pl.ds(ebase, B)], ewb[p], es[p]).wait()

    def wait_scatter(p):
        pltpu.make_async_copy(ewb[p], agg.at[dst_i[p]], ss[p]).wait()

    issue_src(0, 0)
    issue_src(1, 1)
    issue_dst(0, 0)
    wait_src(0)
    issue_gather(0)
    issue_ew(0, 0)

    def step(c, t, p):
        q = 1 - p
        c1 = jnp.minimum(c + 1, CHUNKS - 1)
        c2 = jnp.minimum(c + 2, CHUNKS - 1)
        wait_src(q)
        issue_gather(q)
        if t is None:
            wait_scatter(q)
        else:
            @pl.when(t > 0)
            def _():
                wait_scatter(q)
        issue_dst(c1, q)
        issue_ew(c1, q)
        wait_gather_ew(p)
        issue_src(c2, p)
        wait_dst(p)

        @plsc.parallel_loop(0, B, unroll=4)
        def _(r):
            for g in range(D // (2 * L)):
                xv = xb[p][r, pl.ds(L * g, L)]
                a = lax.bitcast_convert_type(xv << 16, jnp.float32)
                b = lax.bitcast_convert_type(xv & jnp.int32(-65536), jnp.float32)
                s0 = pl.ds(2 * L * g, L)
                s1 = pl.ds(2 * L * g + L, L)
                ewb[p][r, s0] = jnp.maximum(ewb[p][r, s0] + a, 0.0)
                ewb[p][r, s1] = jnp.maximum(ewb[p][r, s1] + b, 0.0)

        pltpu.async_copy(ewb[p], agg.at[dst_i[p]], ss[p], add=True)

    def pair(t, _):
        step(2 * t, t, 0)
        step(2 * t + 1, None, 1)
        return 0

    lax.fori_loop(0, CHUNKS // 2, pair, 0)

    wait_scatter(1)
    wait_gather_ew(0)
    wait_src(1)
    wait_dst(0)

    plsc.subcore_barrier()
    pltpu.sync_copy(agg.at[pl.ds(sid * ROWS_PT, ROWS_PT)],
                    out_hbm.at[cid, pl.ds(sid * ROWS_PT, ROWS_PT)])


@jax.jit
def _sc_aggregate(x, src1, dst1, ew):
    mesh = plsc.VectorSubcoreMesh(core_axis_name="c", subcore_axis_name="s",
                                  num_cores=NC, num_subcores=NS)
    return pl.kernel(
        _sc_edge_kernel,
        out_type=jax.ShapeDtypeStruct((NC, N_PAD, D), jnp.float32),
        mesh=mesh,
        compiler_params=pltpu.CompilerParams(use_tc_tiling_on_sc=False),
        scratch_types=[
            [pltpu.VMEM((B,), jnp.int32)] * NBUF,
            [pltpu.VMEM((B,), jnp.int32)] * NBUF,
            [pltpu.VMEM((B, D), jnp.float32)] * NBUF,
            [pltpu.VMEM((B, D // 2), jnp.int32)] * NBUF,
            pltpu.VMEM_SHARED((N_PAD, D), jnp.float32),
            [pltpu.SemaphoreType.DMA] * NBUF,
            [pltpu.SemaphoreType.DMA] * NBUF,
            [pltpu.SemaphoreType.DMA] * NBUF,
            [pltpu.SemaphoreType.DMA] * NBUF,
            [pltpu.SemaphoreType.DMA] * NBUF,
        ],
    )(x, src1, dst1, ew)


def _tc_mlp_kernel(p0, p1, xb, eps_ref, wnn, bnn, w1, b1, w2, b2, out):
    scale = 1.0 + eps_ref[0]
    h = p0[...] + p1[...] + scale * xb[...]
    h = jnp.maximum(jnp.dot(h, wnn[...], preferred_element_type=jnp.float32)
                    + bnn[...], 0.0)
    h = jnp.maximum(jnp.dot(h, w1[...], preferred_element_type=jnp.float32)
                    + b1[...], 0.0)
    out[...] = jnp.dot(h, w2[...], preferred_element_type=jnp.float32) + b2[...]


@jax.jit
def _tc_mlp(p0, p1, x, eps, W_nn, b_nn, W1, b1, W2, b2):
    R = 2000
    return pl.pallas_call(
        _tc_mlp_kernel,
        grid=(N // R,),
        in_specs=[
            pl.BlockSpec((R, D), lambda i: (i, 0)),
            pl.BlockSpec((R, D), lambda i: (i, 0)),
            pl.BlockSpec((R, D), lambda i: (i, 0)),
            pl.BlockSpec(memory_space=pltpu.SMEM),
            pl.BlockSpec((D, D), lambda i: (0, 0)),
            pl.BlockSpec((1, D), lambda i: (0, 0)),
            pl.BlockSpec((D, D), lambda i: (0, 0)),
            pl.BlockSpec((1, D), lambda i: (0, 0)),
            pl.BlockSpec((D, 1), lambda i: (0, 0)),
            pl.BlockSpec((1, 1), lambda i: (0, 0)),
        ],
        out_specs=pl.BlockSpec((R, 1), lambda i: (i, 0)),
        out_shape=jax.ShapeDtypeStruct((N, 1), jnp.float32),
    )(p0, p1, x, eps, W_nn, b_nn, W1, b1, W2, b2)


def kernel(x, edge_index, edge_weight, eps, W_nn, b_nn, W1, b1, W2, b2):
    xi = (x.reshape(N, D // 32, 2, 16).transpose(0, 1, 3, 2)
          .reshape(N, D).astype(jnp.bfloat16))
    xi = jax.lax.bitcast_convert_type(xi.reshape(N, D // 2, 2), jnp.int32)
    src2 = edge_index[0].reshape(NW * CHUNKS, B)
    dst2 = edge_index[1].reshape(NW * CHUNKS, B)
    partial = _sc_aggregate(xi, src2, dst2, edge_weight)
    return _tc_mlp(partial[0], partial[1], x, eps,
                   W_nn, b_nn.reshape(1, D), W1, b1.reshape(1, D),
                   W2, b2.reshape(1, 1))


# speedup vs baseline: 1.0473x; 1.0473x over previous
import jax
import jax.numpy as jnp
from jax import lax
from jax.experimental import pallas as pl
from jax.experimental.pallas import tpu as pltpu
from jax.experimental.pallas import tpu_sc as plsc

N = 10000
E = 320000
D = 128
L = 16
NC = 2
NS = 16
NW = NC * NS
EPW = E // NW
B = 80
CHUNKS = EPW // B
NBUF = 3
N_PAD = 10240
ROWS_PT = N_PAD // NS


def _sc_edge_kernel(x_hbm, src_hbm, dst_hbm, ew_hbm, out_hbm,
                    src_i, dst_i, ewb, xb, agg,
                    gs, es, ss, isrc, idst):
    cid = lax.axis_index("c")
    sid = lax.axis_index("s")
    wid = cid * NS + sid
    ebase = wid * EPW

    zero = jnp.zeros((L,), jnp.float32)

    def zrow(r, _):
        for j in range(D // L):
            ewb[0][r, pl.ds(j * L, L)] = zero
        return 0

    lax.fori_loop(0, B, zrow, 0)
    for k in range(ROWS_PT // B):
        pltpu.sync_copy(ewb[0], agg.at[pl.ds(sid * ROWS_PT + k * B, B)])
    plsc.subcore_barrier()

    def issue_src(c, p):
        pltpu.async_copy(src_hbm.at[pl.ds(ebase + c * B, B)], src_i[p], isrc[p])

    def issue_dst(c, p):
        pltpu.async_copy(dst_hbm.at[pl.ds(ebase + c * B, B)], dst_i[p], idst[p])

    def issue_gather(p):
        pltpu.async_copy(x_hbm.at[src_i[p]], xb[p], gs[p])

    def issue_ew(c, p):
        pltpu.async_copy(ew_hbm.at[pl.ds(ebase + c * B, B)], ewb[p], es[p])

    def wait_src(p):
        pltpu.make_async_copy(src_hbm.at[pl.ds(ebase, B)], src_i[p],
                              isrc[p]).wait()

    def wait_dst(p):
        pltpu.make_async_copy(dst_hbm.at[pl.ds(ebase, B)], dst_i[p],
                              idst[p]).wait()

    def wait_gather_ew(p):
        pltpu.make_async_copy(x_hbm.at[src_i[p]], xb[p], gs[p]).wait()
        pltpu.make_async_copy(ew_hbm.at[pl.ds(ebase, B)], ewb[p], es[p]).wait()

    def wait_scatter(p):
        pltpu.make_async_copy(ewb[p], agg.at[dst_i[p]], ss[p]).wait()

    issue_src(0, 0)
    issue_src(1, 1)
    issue_dst(0, 0)
    wait_src(0)
    issue_gather(0)
    issue_ew(0, 0)

    def step(c, t, p):
        pn = (p + 1) % NBUF
        pn2 = (p + 2) % NBUF
        c1 = jnp.minimum(c + 1, CHUNKS - 1)
        c2 = jnp.minimum(c + 2, CHUNKS - 1)
        wait_src(pn)
        issue_gather(pn)
        if t is None:
            wait_scatter(pn)
        else:
            @pl.when(t > 0)
            def _():
                wait_scatter(pn)
        issue_dst(c1, pn)
        issue_ew(c1, pn)
        wait_gather_ew(p)
        issue_src(c2, pn2)
        wait_dst(p)

        @plsc.parallel_loop(0, B, unroll=4)
        def _(r):
            for g in range(D // (2 * L)):
                xv = xb[p][r, pl.ds(L * g, L)]
                a = lax.bitcast_convert_type(xv << 16, jnp.float32)
                b = lax.bitcast_convert_type(xv & jnp.int32(-65536), jnp.float32)
                s0 = pl.ds(2 * L * g, L)
                s1 = pl.ds(2 * L * g + L, L)
                ewb[p][r, s0] = jnp.maximum(ewb[p][r, s0] + a, 0.0)
                ewb[p][r, s1] = jnp.maximum(ewb[p][r, s1] + b, 0.0)

        pltpu.async_copy(ewb[p], agg.at[dst_i[p]], ss[p], add=True)

    def triple(t, _):
        step(3 * t, t, 0)
        step(3 * t + 1, t, 1)
        step(3 * t + 2, None, 2)
        return 0

    lax.fori_loop(0, CHUNKS // 3, triple, 0)
    step(jnp.int32(CHUNKS - 2), None, 0)
    step(jnp.int32(CHUNKS - 1), None, 1)

    wait_scatter(0)
    wait_scatter(1)
    wait_gather_ew(2)
    wait_src(0)
    wait_dst(2)

    plsc.subcore_barrier()
    pltpu.sync_copy(agg.at[pl.ds(sid * ROWS_PT, ROWS_PT)],
                    out_hbm.at[cid, pl.ds(sid * ROWS_PT, ROWS_PT)])


@jax.jit
def _sc_aggregate(x, src1, dst1, ew):
    mesh = plsc.VectorSubcoreMesh(core_axis_name="c", subcore_axis_name="s",
                                  num_cores=NC, num_subcores=NS)
    return pl.kernel(
        _sc_edge_kernel,
        out_type=jax.ShapeDtypeStruct((NC, N_PAD, D), jnp.float32),
        mesh=mesh,
        compiler_params=pltpu.CompilerParams(use_tc_tiling_on_sc=False),
        scratch_types=[
            [pltpu.VMEM((B,), jnp.int32)] * NBUF,
            [pltpu.VMEM((B,), jnp.int32)] * NBUF,
            [pltpu.VMEM((B, D), jnp.float32)] * NBUF,
            [pltpu.VMEM((B, D // 2), jnp.int32)] * NBUF,
            pltpu.VMEM_SHARED((N_PAD, D), jnp.float32),
            [pltpu.SemaphoreType.DMA] * NBUF,
            [pltpu.SemaphoreType.DMA] * NBUF,
            [pltpu.SemaphoreType.DMA] * NBUF,
            [pltpu.SemaphoreType.DMA] * NBUF,
            [pltpu.SemaphoreType.DMA] * NBUF,
        ],
    )(x, src1, dst1, ew)


def _tc_mlp_kernel(p0, p1, xb, eps_ref, wnn, bnn, w1, b1, w2, b2, out):
    scale = 1.0 + eps_ref[0]
    h = p0[...] + p1[...] + scale * xb[...]
    h = jnp.maximum(jnp.dot(h, wnn[...], preferred_element_type=jnp.float32)
                    + bnn[...], 0.0)
    h = jnp.maximum(jnp.dot(h, w1[...], preferred_element_type=jnp.float32)
                    + b1[...], 0.0)
    out[...] = jnp.dot(h, w2[...], preferred_element_type=jnp.float32) + b2[...]


@jax.jit
def _tc_mlp(p0, p1, x, eps, W_nn, b_nn, W1, b1, W2, b2):
    R = 2000
    return pl.pallas_call(
        _tc_mlp_kernel,
        grid=(N // R,),
        in_specs=[
            pl.BlockSpec((R, D), lambda i: (i, 0)),
            pl.BlockSpec((R, D), lambda i: (i, 0)),
            pl.BlockSpec((R, D), lambda i: (i, 0)),
            pl.BlockSpec(memory_space=pltpu.SMEM),
            pl.BlockSpec((D, D), lambda i: (0, 0)),
            pl.BlockSpec((1, D), lambda i: (0, 0)),
            pl.BlockSpec((D, D), lambda i: (0, 0)),
            pl.BlockSpec((1, D), lambda i: (0, 0)),
            pl.BlockSpec((D, 1), lambda i: (0, 0)),
            pl.BlockSpec((1, 1), lambda i: (0, 0)),
        ],
        out_specs=pl.BlockSpec((R, 1), lambda i: (i, 0)),
        out_shape=jax.ShapeDtypeStruct((N, 1), jnp.float32),
    )(p0, p1, x, eps, W_nn, b_nn, W1, b1, W2, b2)


def kernel(x, edge_index, edge_weight, eps, W_nn, b_nn, W1, b1, W2, b2):
    xi = (x.reshape(N, D // 32, 2, 16).transpose(0, 1, 3, 2)
          .reshape(N, D).astype(jnp.bfloat16))
    xi = jax.lax.bitcast_convert_type(xi.reshape(N, D // 2, 2), jnp.int32)
    partial = _sc_aggregate(xi, edge_index[0], edge_index[1], edge_weight)
    return _tc_mlp(partial[0], partial[1], x, eps,
                   W_nn, b_nn.reshape(1, D), W1, b1.reshape(1, D),
                   W2, b2.reshape(1, 1))


# speedup vs baseline: 1.0760x; 1.0274x over previous
import jax
import jax.numpy as jnp
from jax import lax
from jax.experimental import pallas as pl
from jax.experimental.pallas import tpu as pltpu
from jax.experimental.pallas import tpu_sc as plsc

N = 10000
E = 320000
D = 128
L = 16
NC = 2
NS = 16
NW = NC * NS
EPW = E // NW
B = 80
CHUNKS = EPW // B
NBUF = 2
N_PAD = 10240
ROWS_PT = N_PAD // NS


def _sc_edge_kernel(x_hbm, src_hbm, dst_hbm, ew_hbm, out_hbm,
                    src_i, dst_i, ewb, xb, agg,
                    gs, es, ss, isrc, idst):
    cid = lax.axis_index("c")
    sid = lax.axis_index("s")
    wid = cid * NS + sid
    ebase = wid * EPW

    zero = jnp.zeros((L,), jnp.float32)

    def zrow(r, _):
        for j in range(D // L):
            ewb[0][r, pl.ds(j * L, L)] = zero
        return 0

    lax.fori_loop(0, B, zrow, 0)
    for k in range(ROWS_PT // B):
        pltpu.sync_copy(ewb[0], agg.at[pl.ds(sid * ROWS_PT + k * B, B)])
    plsc.subcore_barrier()

    def issue_src(c, p):
        pltpu.async_copy(src_hbm.at[pl.ds(ebase + c * B, B)], src_i[p], isrc[p])

    def issue_dst(c, p):
        pltpu.async_copy(dst_hbm.at[pl.ds(ebase + c * B, B)], dst_i[p], idst[p])

    def issue_gather(p):
        pltpu.async_copy(x_hbm.at[src_i[p]], xb[p], gs[p])

    def issue_ew(c, p):
        pltpu.async_copy(ew_hbm.at[pl.ds(ebase + c * B, B)], ewb[p], es[p])

    def wait_src(p):
        pltpu.make_async_copy(src_hbm.at[pl.ds(ebase, B)], src_i[p],
                              isrc[p]).wait()

    def wait_dst(p):
        pltpu.make_async_copy(dst_hbm.at[pl.ds(ebase, B)], dst_i[p],
                              idst[p]).wait()

    def wait_gather_ew(p):
        pltpu.make_async_copy(x_hbm.at[src_i[p]], xb[p], gs[p]).wait()
        pltpu.make_async_copy(ew_hbm.at[pl.ds(ebase, B)], ewb[p], es[p]).wait()

    def wait_scatter(p):
        pltpu.make_async_copy(ewb[p], agg.at[dst_i[p]], ss[p]).wait()

    issue_src(0, 0)
    issue_src(1, 1)
    issue_dst(0, 0)
    wait_src(0)
    issue_gather(0)
    issue_ew(0, 0)

    def step(c, t, p):
        q = 1 - p
        c1 = jnp.minimum(c + 1, CHUNKS - 1)
        c2 = jnp.minimum(c + 2, CHUNKS - 1)
        wait_src(q)
        issue_gather(q)
        if t is None:
            wait_scatter(q)
        else:
            @pl.when(t > 0)
            def _():
                wait_scatter(q)
        issue_dst(c1, q)
        issue_ew(c1, q)
        wait_gather_ew(p)
        issue_src(c2, p)
        wait_dst(p)

        @plsc.parallel_loop(0, B, unroll=4)
        def _(r):
            for g in range(D // (2 * L)):
                xv = xb[p][r, pl.ds(L * g, L)]
                a = lax.bitcast_convert_type(xv << 16, jnp.float32)
                b = lax.bitcast_convert_type(xv & jnp.int32(-65536), jnp.float32)
                s0 = pl.ds(2 * L * g, L)
                s1 = pl.ds(2 * L * g + L, L)
                ewb[p][r, s0] = jnp.maximum(ewb[p][r, s0] + a, 0.0)
                ewb[p][r, s1] = jnp.maximum(ewb[p][r, s1] + b, 0.0)

        pltpu.async_copy(ewb[p], agg.at[dst_i[p]], ss[p], add=True)

    def pair(t, _):
        step(2 * t, t, 0)
        step(2 * t + 1, None, 1)
        return 0

    lax.fori_loop(0, CHUNKS // 2, pair, 0)
    step(jnp.int32(CHUNKS - 1), None, 0)

    wait_scatter(0)
    wait_gather_ew(1)
    wait_src(0)
    wait_dst(1)

    plsc.subcore_barrier()
    pltpu.sync_copy(agg.at[pl.ds(sid * ROWS_PT, ROWS_PT)],
                    out_hbm.at[cid, pl.ds(sid * ROWS_PT, ROWS_PT)])


@jax.jit
def _sc_aggregate(x, src1, dst1, ew):
    mesh = plsc.VectorSubcoreMesh(core_axis_name="c", subcore_axis_name="s",
                                  num_cores=NC, num_subcores=NS)
    return pl.kernel(
        _sc_edge_kernel,
        out_type=jax.ShapeDtypeStruct((NC, N_PAD, D), jnp.float32),
        mesh=mesh,
        compiler_params=pltpu.CompilerParams(use_tc_tiling_on_sc=False),
        scratch_types=[
            [pltpu.VMEM((B,), jnp.int32)] * NBUF,
            [pltpu.VMEM((B,), jnp.int32)] * NBUF,
            [pltpu.VMEM((B, D), jnp.float32)] * NBUF,
            [pltpu.VMEM((B, D // 2), jnp.int32)] * NBUF,
            pltpu.VMEM_SHARED((N_PAD, D), jnp.float32),
            [pltpu.SemaphoreType.DMA] * NBUF,
            [pltpu.SemaphoreType.DMA] * NBUF,
            [pltpu.SemaphoreType.DMA] * NBUF,
            [pltpu.SemaphoreType.DMA] * NBUF,
            [pltpu.SemaphoreType.DMA] * NBUF,
        ],
    )(x, src1, dst1, ew)


def _tc_mlp_kernel(p0, p1, xb, eps_ref, wnn, bnn, w1, b1, w2, b2, out):
    scale = 1.0 + eps_ref[0]
    h = p0[...] + p1[...] + scale * xb[...]
    h = jnp.maximum(jnp.dot(h, wnn[...], preferred_element_type=jnp.float32)
                    + bnn[...], 0.0)
    h = jnp.maximum(jnp.dot(h, w1[...], preferred_element_type=jnp.float32)
                    + b1[...], 0.0)
    out[...] = jnp.dot(h, w2[...], preferred_element_type=jnp.float32) + b2[...]


@jax.jit
def _tc_mlp(p0, p1, x, eps, W_nn, b_nn, W1, b1, W2, b2):
    R = 2000
    return pl.pallas_call(
        _tc_mlp_kernel,
        grid=(N // R,),
        in_specs=[
            pl.BlockSpec((R, D), lambda i: (i, 0)),
            pl.BlockSpec((R, D), lambda i: (i, 0)),
            pl.BlockSpec((R, D), lambda i: (i, 0)),
            pl.BlockSpec(memory_space=pltpu.SMEM),
            pl.BlockSpec((D, D), lambda i: (0, 0)),
            pl.BlockSpec((1, D), lambda i: (0, 0)),
            pl.BlockSpec((D, D), lambda i: (0, 0)),
            pl.BlockSpec((1, D), lambda i: (0, 0)),
            pl.BlockSpec((D, 1), lambda i: (0, 0)),
            pl.BlockSpec((1, 1), lambda i: (0, 0)),
        ],
        out_specs=pl.BlockSpec((R, 1), lambda i: (i, 0)),
        out_shape=jax.ShapeDtypeStruct((N, 1), jnp.float32),
    )(p0, p1, x, eps, W_nn, b_nn, W1, b1, W2, b2)


def kernel(x, edge_index, edge_weight, eps, W_nn, b_nn, W1, b1, W2, b2):
    xi = (x.reshape(N, D // 32, 2, 16).transpose(0, 1, 3, 2)
          .reshape(N, D).astype(jnp.bfloat16))
    xi = jax.lax.bitcast_convert_type(xi.reshape(N, D // 2, 2), jnp.int32)
    partial = _sc_aggregate(xi, edge_index[0], edge_index[1], edge_weight)
    return _tc_mlp(partial[0], partial[1], x, eps,
                   W_nn, b_nn.reshape(1, D), W1, b1.reshape(1, D),
                   W2, b2.reshape(1, 1))
